# baseline (speedup 1.0000x reference)
import functools

import jax
import jax.numpy as jnp
from jax import lax
from jax.experimental import pallas as pl
from jax.experimental.pallas import tpu as pltpu
from jax.experimental.pallas import tpu_sc as plsc

B = 16384
F = 32
N_ROWS = 1000000
NC = 2
NS = 16
L = 16
NW = NC * NS
BPW = B // NW
CHUNKS = BPW // L

_mesh = plsc.VectorSubcoreMesh(core_axis_name="c", subcore_axis_name="s")


@functools.partial(
    pl.kernel,
    mesh=_mesh,
    out_type=jax.ShapeDtypeStruct((B,), jnp.float32),
    compiler_params=pltpu.CompilerParams(
        needs_layout_passes=False, use_tc_tiling_on_sc=False),
    scratch_types=[
        pltpu.VMEM((F, BPW), jnp.int32),
        pltpu.VMEM((F, BPW), jnp.int32),
        pltpu.VMEM((BPW,), jnp.int32),
        pltpu.VMEM((BPW,), jnp.int32),
        pltpu.VMEM((F, BPW), jnp.float32),
        pltpu.VMEM((F, BPW), jnp.float32),
        pltpu.VMEM((BPW, 1), jnp.float32),
        pltpu.VMEM((BPW, 1), jnp.float32),
        pltpu.VMEM((L,), jnp.float32),
        pltpu.VMEM((BPW,), jnp.float32),
        pltpu.SemaphoreType.DMA,
    ],
)
def _mf_kernel(uidx_hbm, iidx_hbm, uid_hbm, iid_hbm, uf_hbm, if_hbm,
               ub_hbm, ib_hbm, gb_hbm,
               out_hbm,
               uidx_v, iidx_v, uid_v, iid_v, u_cols, i_cols, ub_v, ib_v,
               gb_v, out_v, sem):
    wid = lax.axis_index("s") * NC + lax.axis_index("c")
    base = wid * BPW

    pltpu.sync_copy(uidx_hbm.at[wid], uidx_v)
    pltpu.sync_copy(iidx_hbm.at[wid], iidx_v)
    pltpu.sync_copy(uid_hbm.at[pl.ds(base, BPW)], uid_v)
    pltpu.sync_copy(iid_hbm.at[pl.ds(base, BPW)], iid_v)

    copies = []
    for f in range(F):
        copies.append(
            pltpu.async_copy(uf_hbm.at[uidx_v.at[f]], u_cols.at[f], sem))
        copies.append(
            pltpu.async_copy(if_hbm.at[iidx_v.at[f]], i_cols.at[f], sem))
    cub = pltpu.async_copy(ub_hbm.at[uid_v], ub_v, sem)
    cib = pltpu.async_copy(ib_hbm.at[iid_v], ib_v, sem)
    pltpu.sync_copy(gb_hbm.at[...], gb_v)
    for c in copies:
        c.wait()
    cub.wait()
    cib.wait()

    gb = gb_v[...]
    lane = lax.iota(jnp.int32, L)
    zero = jnp.full((L,), 0, jnp.int32)
    for c in range(CHUNKS):
        rows = jnp.full((L,), c * L, jnp.int32) + lane
        acc = (plsc.load_gather(ub_v, [rows, zero])
               + plsc.load_gather(ib_v, [rows, zero]) + gb)
        for f in range(F):
            acc = (acc
                   + u_cols[f, pl.ds(c * L, L)] * i_cols[f, pl.ds(c * L, L)])
        out_v[pl.ds(c * L, L)] = acc

    pltpu.sync_copy(out_v, out_hbm.at[pl.ds(base, BPW)])


def kernel(user_ids, item_ids, user_factors, item_factors, user_bias,
           item_bias, global_bias):
    uid = user_ids.astype(jnp.int32)
    iid = item_ids.astype(jnp.int32)
    foffs = (jnp.arange(F, dtype=jnp.int32) * N_ROWS).reshape(1, F, 1)
    uidx = uid.reshape(NW, 1, BPW) + foffs
    iidx = iid.reshape(NW, 1, BPW) + foffs
    uf = user_factors.T.reshape(-1)
    itf = item_factors.T.reshape(-1)
    gb = jnp.broadcast_to(global_bias.astype(jnp.float32), (L,))
    return _mf_kernel(uidx, iidx, uid, iid, uf, itf, user_bias, item_bias,
                      gb)
